# baseline (speedup 1.0000x reference)
import jax
import jax.numpy as jnp
from jax import lax
from jax.experimental import pallas as pl
from jax.experimental.pallas import tpu as pltpu
from jax.experimental.pallas import tpu_sc as plsc

VSZ = 1000000
DSZ = 64
B = 4096
L = 50

NC = 2
NS = 16
NW = NC * NS

ROWS_W = B // NW
NBUF = 4


def _body(x_hbm, table_hbm, out_hbm, idx_v, rows_v, *sems):
    gsem = sems[:NBUF]
    ssem = sems[NBUF:]
    wid = lax.axis_index("s") * NC + lax.axis_index("c")
    r0 = wid * ROWS_W
    pltpu.sync_copy(x_hbm.at[pl.ds(r0, ROWS_W)], idx_v)

    for b in range(NBUF):
        pltpu.async_copy(table_hbm.at[idx_v.at[b]], rows_v.at[b], gsem[b])

    @pl.loop(0, ROWS_W - NBUF, step=NBUF)
    def _(i):
        for b in range(NBUF):
            r = i + b
            pltpu.make_async_copy(
                table_hbm.at[idx_v.at[r]], rows_v.at[b], gsem[b]
            ).wait()
            pltpu.async_copy(rows_v.at[b], out_hbm.at[r0 + r], ssem[b])
        for b in range(NBUF):
            r = i + b
            pltpu.make_async_copy(
                rows_v.at[b], out_hbm.at[r0 + r], ssem[b]
            ).wait()
            pltpu.async_copy(
                table_hbm.at[idx_v.at[r + NBUF]], rows_v.at[b], gsem[b]
            )

    for b in range(NBUF):
        r = ROWS_W - NBUF + b
        pltpu.make_async_copy(
            table_hbm.at[idx_v.at[r]], rows_v.at[b], gsem[b]
        ).wait()
        pltpu.async_copy(rows_v.at[b], out_hbm.at[r0 + r], ssem[b])
    for b in range(NBUF):
        r = ROWS_W - NBUF + b
        pltpu.make_async_copy(rows_v.at[b], out_hbm.at[r0 + r], ssem[b]).wait()


@jax.jit
def _lookup(x, table):
    mesh = plsc.VectorSubcoreMesh(core_axis_name="c", subcore_axis_name="s")
    return pl.kernel(
        _body,
        out_type=jax.ShapeDtypeStruct((B, L, DSZ), jnp.float32),
        mesh=mesh,
        scratch_types=[
            pltpu.VMEM((ROWS_W, L), jnp.int32),
            pltpu.VMEM((NBUF, L, DSZ), jnp.float32),
        ]
        + [pltpu.SemaphoreType.DMA] * (2 * NBUF),
        compiler_params=pltpu.CompilerParams(use_tc_tiling_on_sc=False),
    )(x, table)


def kernel(x, table):
    return _lookup(x, table)
